# baseline (speedup 1.0000x reference)
import functools

import jax
import jax.numpy as jnp
from jax import lax
from jax.experimental import pallas as pl
from jax.experimental.pallas import tpu as pltpu
from jax.experimental.pallas import tpu_sc as plsc

_THRESH = 1.0
_LANES = 16

_NC, _NS = 2, 16
_NW = _NC * _NS


def _make_kernel(B, L, H, b_per_w):
    mesh = plsc.VectorSubcoreMesh(core_axis_name="c", subcore_axis_name="s")

    @functools.partial(
        pl.kernel,
        mesh=mesh,
        out_type=jax.ShapeDtypeStruct((B, L, H), jnp.float32),
        scratch_types=[
            pltpu.VMEM((b_per_w,), jnp.int32),
            pltpu.VMEM((b_per_w, L, H), jnp.float32),
            pltpu.VMEM((b_per_w, L, H), jnp.float32),
            pltpu.SemaphoreType.DMA,
            pltpu.SemaphoreType.DMA,
        ],
        compiler_params=pltpu.CompilerParams(use_tc_tiling_on_sc=False),
    )
    def mask_mul(x_hbm, tab_hbm, idx_hbm, out_hbm, idx_v, rows_v, x_v, g_sem, x_sem):
        wid = lax.axis_index("s") * _NC + lax.axis_index("c")
        base = wid * b_per_w
        pltpu.sync_copy(idx_hbm.at[pl.ds(base, b_per_w)], idx_v)
        gather = pltpu.async_copy(tab_hbm.at[idx_v], rows_v, g_sem)
        xload = pltpu.async_copy(x_hbm.at[pl.ds(base, b_per_w)], x_v, x_sem)
        gather.wait()
        xload.wait()

        def row_body(r, carry):
            for l in range(L):
                e = rows_v[r, l]
                xv = x_v[r, l]
                rows_v[r, l] = jnp.where(e <= _THRESH, xv, 0.0)
            return carry

        lax.fori_loop(0, b_per_w, row_body, 0)
        pltpu.sync_copy(rows_v, out_hbm.at[pl.ds(base, b_per_w)])

    return mask_mul


@jax.jit
def kernel(x, energy_local, batch_idx):
    B, L, H = x.shape
    idx = batch_idx.astype(jnp.int32)
    return _make_kernel(B, L, H, B // _NW)(x, energy_local, idx)


# speedup vs baseline: 11.4238x; 11.4238x over previous
import functools

import jax
import jax.numpy as jnp
from jax import lax
from jax.experimental import pallas as pl
from jax.experimental.pallas import tpu as pltpu
from jax.experimental.pallas import tpu_sc as plsc

_THRESH = 1.0
_LANES = 16

_NC, _NS = 2, 16
_NW = _NC * _NS


def _make_kernel(B, L, H, N):
    planes_per_w = (L * H) // _NW
    n_vecs = B // _LANES
    mesh = plsc.VectorSubcoreMesh(core_axis_name="c", subcore_axis_name="s")

    @functools.partial(
        pl.kernel,
        mesh=mesh,
        out_type=jax.ShapeDtypeStruct((L, H, B), jnp.float32),
        scratch_types=[
            pltpu.VMEM((B,), jnp.int32),
            pltpu.VMEM((N,), jnp.float32),
            pltpu.VMEM((B,), jnp.float32),
            pltpu.VMEM((B,), jnp.float32),
            pltpu.SemaphoreType.DMA,
            pltpu.SemaphoreType.DMA,
        ],
        compiler_params=pltpu.CompilerParams(needs_layout_passes=False),
    )
    def mask_mul(x_hbm, tab_hbm, idx_hbm, out_hbm, idx_v, plane_v, x_v, o_v, t_sem, x_sem):
        wid = lax.axis_index("s") * _NC + lax.axis_index("c")
        pltpu.sync_copy(idx_hbm, idx_v)

        def plane_body(k, carry):
            p = wid * planes_per_w + k
            l = p // H
            h = p % H
            t_copy = pltpu.async_copy(tab_hbm.at[l, h], plane_v, t_sem)
            x_copy = pltpu.async_copy(x_hbm.at[l, h], x_v, x_sem)
            t_copy.wait()
            x_copy.wait()

            def vec_body(i, c2):
                sl = pl.ds(i * _LANES, _LANES)
                iv = idx_v[sl]
                e = plsc.load_gather(plane_v, [iv])
                xv = x_v[sl]
                o_v[sl] = jnp.where(e <= _THRESH, xv, 0.0)
                return c2

            lax.fori_loop(0, n_vecs, vec_body, 0)
            pltpu.sync_copy(o_v, out_hbm.at[l, h])
            return carry

        lax.fori_loop(0, planes_per_w, plane_body, 0)

    return mask_mul


@jax.jit
def kernel(x, energy_local, batch_idx):
    B, L, H = x.shape
    N = energy_local.shape[0]
    x_t = x.transpose(1, 2, 0)
    tab_t = energy_local.transpose(1, 2, 0)
    idx = batch_idx.astype(jnp.int32)
    out_t = _make_kernel(B, L, H, N)(x_t, tab_t, idx)
    return out_t.transpose(2, 0, 1)


# speedup vs baseline: 13.0177x; 1.1395x over previous
import functools

import jax
import jax.numpy as jnp
from jax import lax
from jax.experimental import pallas as pl
from jax.experimental.pallas import tpu as pltpu
from jax.experimental.pallas import tpu_sc as plsc

_THRESH = 1.0
_LANES = 16

_NC, _NS = 2, 16
_NW = _NC * _NS

_TILE_MINOR = 128


def _make_kernel(B, L, H, N):
    planes_per_w = (L * H) // _NW
    n_vecs = B // _LANES
    half1 = ((N // _TILE_MINOR) // 2) * _TILE_MINOR
    half2 = N - half1
    mesh = plsc.VectorSubcoreMesh(core_axis_name="c", subcore_axis_name="s")

    @functools.partial(
        pl.kernel,
        mesh=mesh,
        out_type=jax.ShapeDtypeStruct((L, H, B), jnp.float32),
        scratch_types=[
            pltpu.VMEM((B,), jnp.int32),
            pltpu.VMEM((half1,), jnp.float32),
            pltpu.VMEM((half2,), jnp.float32),
            pltpu.VMEM((B,), jnp.float32),
            pltpu.VMEM((B,), jnp.float32),
            pltpu.SemaphoreType.DMA,
            pltpu.SemaphoreType.DMA,
            pltpu.SemaphoreType.DMA,
        ],
        compiler_params=pltpu.CompilerParams(needs_layout_passes=False),
    )
    def mask_mul(x_hbm, tab_hbm, idx_hbm, out_hbm, idx_v, buf_a, buf_b, x_v, o_v,
                 sem_a, sem_b, sem_x):
        wid = lax.axis_index("s") * _NC + lax.axis_index("c")
        p0 = wid * planes_per_w
        pltpu.sync_copy(idx_hbm, idx_v)

        def copy_a(p):
            return pltpu.make_async_copy(
                tab_hbm.at[p // H, p % H, pl.ds(0, half1)], buf_a, sem_a)

        def copy_b(p):
            return pltpu.make_async_copy(
                tab_hbm.at[p // H, p % H, pl.ds(half1, half2)], buf_b, sem_b)

        def copy_x(p):
            return pltpu.make_async_copy(x_hbm.at[p // H, p % H], x_v, sem_x)

        copy_a(p0).start()
        copy_b(p0).start()
        copy_x(p0).start()

        def plane_body(k, carry):
            p = p0 + k
            copy_a(p).wait()
            copy_x(p).wait()

            def vec_a(i, c2):
                for u in range(4):
                    sl = pl.ds((i * 4 + u) * _LANES, _LANES)
                    iv = idx_v[sl]
                    m = iv < half1
                    ivc = jnp.where(m, iv, 0)
                    e = plsc.load_gather(buf_a, [ivc])
                    xv = x_v[sl]
                    o_v[sl] = jnp.where(m & (e <= _THRESH), xv, 0.0)
                return c2

            lax.fori_loop(0, n_vecs // 4, vec_a, 0)

            @pl.when(k < planes_per_w - 1)
            def _():
                copy_a(p + 1).start()

            copy_b(p).wait()

            def vec_b(i, c2):
                for u in range(4):
                    sl = pl.ds((i * 4 + u) * _LANES, _LANES)
                    iv = idx_v[sl]
                    m = iv >= half1
                    ivc = jnp.where(m, iv - half1, 0)
                    e = plsc.load_gather(buf_b, [ivc])
                    xv = x_v[sl]
                    o_v[sl] = o_v[sl] + jnp.where(m & (e <= _THRESH), xv, 0.0)
                return c2

            lax.fori_loop(0, n_vecs // 4, vec_b, 0)

            @pl.when(k < planes_per_w - 1)
            def _():
                copy_b(p + 1).start()
                copy_x(p + 1).start()

            pltpu.sync_copy(o_v, out_hbm.at[p // H, p % H])
            return carry

        lax.fori_loop(0, planes_per_w, plane_body, 0)

    return mask_mul


@jax.jit
def kernel(x, energy_local, batch_idx):
    B, L, H = x.shape
    N = energy_local.shape[0]
    x_t = x.transpose(1, 2, 0)
    tab_t = energy_local.transpose(1, 2, 0)
    idx = batch_idx.astype(jnp.int32)
    out_t = _make_kernel(B, L, H, N)(x_t, tab_t, idx)
    return out_t.transpose(2, 0, 1)
